# Initial kernel scaffold; baseline (speedup 1.0000x reference)
import functools
import jax
import jax.numpy as jnp
from jax import lax
from jax.experimental import pallas as pl
from jax.experimental.pallas import tpu as pltpu
from jax.experimental.pallas import tpu_sc as plsc

N = 10000
E = 320000
D = 128
NP = 10240
NC = 2
NS = 16
L = 16
NW = NC * NS
EPT = E // NW
CH = 80
NCHUNK = EPT // CH
ROWS_PT = NP // NS

_f32 = jnp.float32


def _proj_body(x_ref, w_ref, a_ref, h_ref, al_ref):
    h = jnp.dot(x_ref[...], w_ref[...], preferred_element_type=_f32)
    h_ref[...] = h
    al_ref[...] = jnp.dot(h, a_ref[...], preferred_element_type=_f32)


def _proj(x, w, a_pad, blk=1024):
    grid = NP // blk
    return pl.pallas_call(
        _proj_body,
        grid=(grid,),
        in_specs=[
            pl.BlockSpec((blk, D), lambda i: (i, 0)),
            pl.BlockSpec((D, D), lambda i: (0, 0)),
            pl.BlockSpec((D, 128), lambda i: (0, 0)),
        ],
        out_specs=[
            pl.BlockSpec((blk, D), lambda i: (i, 0)),
            pl.BlockSpec((blk, 128), lambda i: (i, 0)),
        ],
        out_shape=[
            jax.ShapeDtypeStruct((NP, D), _f32),
            jax.ShapeDtypeStruct((NP, 128), _f32),
        ],
    )(x, w, a_pad)


def _combine_body(p_ref, den_ref, o_ref):
    den = den_ref[0] + den_ref[1]
    inv = 1.0 / (den + 1e-16)
    o_ref[...] = jnp.maximum((p_ref[0] + p_ref[1]) * inv, 0.0)


def _combine(p, den, blk=1024):
    grid = NP // blk
    return pl.pallas_call(
        _combine_body,
        grid=(grid,),
        in_specs=[
            pl.BlockSpec((2, blk, D), lambda i: (0, i, 0)),
            pl.BlockSpec((2, blk, 1), lambda i: (0, i, 0)),
        ],
        out_specs=pl.BlockSpec((blk, D), lambda i: (i, 0)),
        out_shape=jax.ShapeDtypeStruct((NP, D), _f32),
    )(p, den)


def _edge_body(src_hbm, dst_hbm, h_hbm, as_hbm, ad_hbm,
               outp_hbm, denp_hbm,
               as_v, ad_v, den_v, acc_v, tmp_v, srcb, dstb, wb, rows_v,
               outp_s, stage_s, sem):
    c = lax.axis_index("c")
    s = lax.axis_index("s")
    wid = c * NS + s
    ebase = wid * EPT

    pltpu.sync_copy(as_hbm, as_v)
    pltpu.sync_copy(ad_hbm, ad_v)

    def _zd(i, _):
        den_v[pl.ds(i * L, L)] = jnp.zeros((L,), _f32)
        return 0
    lax.fori_loop(0, NP // L, _zd, 0)
    for i in range(ROWS_PT // L):
        acc_v[pl.ds(i * L, L)] = jnp.zeros((L,), _f32)

    def _zr(i, _):
        for j in range(D // L):
            rows_v[i, pl.ds(j * L, L)] = jnp.zeros((L,), _f32)
        return 0
    lax.fori_loop(0, CH, _zr, 0)
    for k in range(ROWS_PT // CH):
        pltpu.sync_copy(rows_v, outp_s.at[pl.ds(s * ROWS_PT + k * CH, CH), :])
    plsc.subcore_barrier()

    def _chunk(j, _):
        base = ebase + j * CH
        pltpu.sync_copy(src_hbm.at[pl.ds(base, CH)], srcb)
        pltpu.sync_copy(dst_hbm.at[pl.ds(base, CH)], dstb)
        pltpu.async_copy(h_hbm.at[srcb], rows_v, sem).wait()
        for g in range(CH // L):
            si = srcb[pl.ds(g * L, L)]
            di = dstb[pl.ds(g * L, L)]
            e = plsc.load_gather(as_v, [si]) + plsc.load_gather(ad_v, [di])
            e = jnp.maximum(e, 0.2 * e)
            w = jnp.exp(e)
            wb[pl.ds(g * L, L)] = w
            plsc.addupdate_scatter(den_v, [di], w)
        for e_i in range(CH):
            bc = plsc.load_gather(wb, [jnp.full((L,), e_i, jnp.int32)])
            for j2 in range(D // L):
                sl = pl.ds(j2 * L, L)
                rows_v[e_i, sl] = rows_v[e_i, sl] * bc
        pltpu.sync_copy(rows_v, outp_s.at[dstb], add=True)
        return 0
    lax.fori_loop(0, NCHUNK, _chunk, 0)

    pltpu.sync_copy(den_v, stage_s.at[s])
    plsc.subcore_barrier()
    rbase = s * ROWS_PT
    def _red(k, _):
        pltpu.sync_copy(stage_s.at[k, pl.ds(rbase, ROWS_PT)], tmp_v)
        for i in range(ROWS_PT // L):
            sl = pl.ds(i * L, L)
            acc_v[sl] = acc_v[sl] + tmp_v[sl]
        return 0
    lax.fori_loop(0, NS, _red, 0)
    pltpu.sync_copy(acc_v, denp_hbm.at[c, pl.ds(rbase, ROWS_PT)])

    pltpu.sync_copy(outp_s.at[pl.ds(rbase, ROWS_PT), :],
                    outp_hbm.at[c, pl.ds(rbase, ROWS_PT), :])


def _edge_pass(src, dst, h, a_s, a_d):
    mesh = plsc.VectorSubcoreMesh(core_axis_name="c", subcore_axis_name="s")
    kern = pl.kernel(
        _edge_body,
        out_type=[
            jax.ShapeDtypeStruct((NC, NP, D), _f32),
            jax.ShapeDtypeStruct((NC, NP), _f32),
        ],
        mesh=mesh,
        scratch_types=[
            pltpu.VMEM((NP,), _f32),
            pltpu.VMEM((NP,), _f32),
            pltpu.VMEM((NP,), _f32),
            pltpu.VMEM((ROWS_PT,), _f32),
            pltpu.VMEM((ROWS_PT,), _f32),
            pltpu.VMEM((CH,), jnp.int32),
            pltpu.VMEM((CH,), jnp.int32),
            pltpu.VMEM((CH,), _f32),
            pltpu.VMEM((CH, D), _f32),
            pltpu.VMEM_SHARED((NP, D), _f32),
            pltpu.VMEM_SHARED((NS, NP), _f32),
            pltpu.SemaphoreType.DMA,
        ],
    )
    return kern(src, dst, h, a_s, a_d)


@jax.jit
def kernel(x, edge_index, W1, a_src1, a_dst1, W2, a_src2, a_dst2):
    src = edge_index[0]
    dst = edge_index[1]
    xp = jnp.pad(x, ((0, NP - N), (0, 0)))

    def a_pad(a_s, a_d):
        ap = jnp.zeros((D, 128), _f32)
        return ap.at[:, 0].set(a_s).at[:, 1].set(a_d)

    h1, al1 = _proj(xp, W1, a_pad(a_src1, a_dst1))
    p1, den1 = _edge_pass(src, dst, h1, al1[:, 0], al1[:, 1])
    y1 = _combine(p1, den1[:, :, None])
    h2, al2 = _proj(y1, W2, a_pad(a_src2, a_dst2))
    p2, den2 = _edge_pass(src, dst, h2, al2[:, 0], al2[:, 1])
    y2 = _combine(p2, den2[:, :, None])
    return y2[:N]


# speedup vs baseline: 23.5822x; 23.5822x over previous
import functools
import jax
import jax.numpy as jnp
from jax import lax
from jax.experimental import pallas as pl
from jax.experimental.pallas import tpu as pltpu
from jax.experimental.pallas import tpu_sc as plsc

N = 10000
E = 320000
D = 128
NP = 10240
NC = 2
NS = 16
L = 16
NW = NC * NS
EPT = E // NW
CH = 80
NCHUNK = EPT // CH
ROWS_PT = NP // NS

_f32 = jnp.float32


def _proj_body(x_ref, w_ref, a_ref, h_ref, al_ref):
    h = jnp.dot(x_ref[...], w_ref[...], preferred_element_type=_f32)
    h_ref[...] = h
    al_ref[...] = jnp.dot(h, a_ref[...], preferred_element_type=_f32)


def _proj(x, w, a_pad, blk=1024):
    grid = NP // blk
    return pl.pallas_call(
        _proj_body,
        grid=(grid,),
        in_specs=[
            pl.BlockSpec((blk, D), lambda i: (i, 0)),
            pl.BlockSpec((D, D), lambda i: (0, 0)),
            pl.BlockSpec((D, 128), lambda i: (0, 0)),
        ],
        out_specs=[
            pl.BlockSpec((blk, D), lambda i: (i, 0)),
            pl.BlockSpec((blk, 128), lambda i: (i, 0)),
        ],
        out_shape=[
            jax.ShapeDtypeStruct((NP, D), _f32),
            jax.ShapeDtypeStruct((NP, 128), _f32),
        ],
    )(x, w, a_pad)


def _combine_body(p_ref, den_ref, o_ref):
    den = jnp.sum(den_ref[...], axis=0)
    inv = 1.0 / (den + 1e-16)
    o_ref[...] = jnp.maximum((p_ref[0] + p_ref[1]) * inv, 0.0)


def _combine(p, den, blk=1024):
    grid = NP // blk
    return pl.pallas_call(
        _combine_body,
        grid=(grid,),
        in_specs=[
            pl.BlockSpec((2, blk, D), lambda i: (0, i, 0)),
            pl.BlockSpec((NC, blk, 1), lambda i: (0, i, 0)),
        ],
        out_specs=pl.BlockSpec((blk, D), lambda i: (i, 0)),
        out_shape=jax.ShapeDtypeStruct((NP, D), _f32),
    )(p, den)


def _edge_body(src_hbm, dst_hbm, h_hbm, as_hbm, ad_hbm,
               outp_hbm, denp_hbm,
               as_v, ad_v, srcb, dstb, wb, rows_v,
               outp_s, den_s, sem):
    c = lax.axis_index("c")
    s = lax.axis_index("s")
    wid = c * NS + s
    ebase = wid * EPT

    pltpu.sync_copy(as_hbm, as_v)
    pltpu.sync_copy(ad_hbm, ad_v)

    def _zr(i, _):
        for j in range(D // L):
            rows_v[i, pl.ds(j * L, L)] = jnp.zeros((L,), _f32)
        return 0
    lax.fori_loop(0, CH, _zr, 0)
    for g in range(CH // L):
        wb[pl.ds(g * L, L)] = jnp.zeros((L,), _f32)
    for k in range(ROWS_PT // CH):
        pltpu.sync_copy(rows_v, outp_s.at[pl.ds(s * ROWS_PT + k * CH, CH), :])
        pltpu.sync_copy(wb, den_s.at[pl.ds(s * ROWS_PT + k * CH, CH)])
    plsc.subcore_barrier()

    def _chunk(j, _):
        base = ebase + j * CH
        pltpu.sync_copy(src_hbm.at[pl.ds(base, CH)], srcb)
        pltpu.sync_copy(dst_hbm.at[pl.ds(base, CH)], dstb)
        pltpu.async_copy(h_hbm.at[srcb], rows_v, sem).wait()
        for g in range(CH // L):
            si = srcb[pl.ds(g * L, L)]
            di = dstb[pl.ds(g * L, L)]
            e = plsc.load_gather(as_v, [si]) + plsc.load_gather(ad_v, [di])
            e = jnp.maximum(e, 0.2 * e)
            w = jnp.exp(e)
            wb[pl.ds(g * L, L)] = w
            for ei in range(L):
                r = g * L + ei
                ws = w[ei]
                for j2 in range(D // L):
                    sl = pl.ds(j2 * L, L)
                    rows_v[r, sl] = rows_v[r, sl] * ws
        pltpu.sync_copy(rows_v, outp_s.at[dstb], add=True)
        pltpu.sync_copy(wb, den_s.at[dstb], add=True)
        return 0
    lax.fori_loop(0, NCHUNK, _chunk, 0)
    plsc.subcore_barrier()

    rbase = s * ROWS_PT
    pltpu.sync_copy(outp_s.at[pl.ds(rbase, ROWS_PT), :],
                    outp_hbm.at[c, pl.ds(rbase, ROWS_PT), :])
    pltpu.sync_copy(den_s.at[pl.ds(rbase, ROWS_PT)],
                    denp_hbm.at[c, pl.ds(rbase, ROWS_PT)])


def _edge_pass(src, dst, h, a_s, a_d):
    mesh = plsc.VectorSubcoreMesh(core_axis_name="c", subcore_axis_name="s")
    kern = pl.kernel(
        _edge_body,
        out_type=[
            jax.ShapeDtypeStruct((NC, NP, D), _f32),
            jax.ShapeDtypeStruct((NC, NP), _f32),
        ],
        mesh=mesh,
        scratch_types=[
            pltpu.VMEM((NP,), _f32),
            pltpu.VMEM((NP,), _f32),
            pltpu.VMEM((CH,), jnp.int32),
            pltpu.VMEM((CH,), jnp.int32),
            pltpu.VMEM((CH,), _f32),
            pltpu.VMEM((CH, D), _f32),
            pltpu.VMEM_SHARED((NP, D), _f32),
            pltpu.VMEM_SHARED((NP,), _f32),
            pltpu.SemaphoreType.DMA,
        ],
        compiler_params=pltpu.CompilerParams(needs_layout_passes=False),
    )
    return kern(src, dst, h, a_s, a_d)


@jax.jit
def kernel(x, edge_index, W1, a_src1, a_dst1, W2, a_src2, a_dst2):
    src = edge_index[0]
    dst = edge_index[1]
    xp = jnp.pad(x, ((0, NP - N), (0, 0)))

    def a_pad(a_s, a_d):
        ap = jnp.zeros((D, 128), _f32)
        return ap.at[:, 0].set(a_s).at[:, 1].set(a_d)

    h1, al1 = _proj(xp, W1, a_pad(a_src1, a_dst1))
    p1, den1 = _edge_pass(src, dst, h1, al1[:, 0], al1[:, 1])
    y1 = _combine(p1, den1[:, :, None])
    h2, al2 = _proj(y1, W2, a_pad(a_src2, a_dst2))
    p2, den2 = _edge_pass(src, dst, h2, al2[:, 0], al2[:, 1])
    y2 = _combine(p2, den2[:, :, None])
    return y2[:N]


# speedup vs baseline: 27.9571x; 1.1855x over previous
import functools
import jax
import jax.numpy as jnp
from jax import lax
from jax.experimental import pallas as pl
from jax.experimental.pallas import tpu as pltpu
from jax.experimental.pallas import tpu_sc as plsc

N = 10000
E = 320000
D = 128
NP = 10240
NC = 2
NS = 16
L = 16
NW = NC * NS
EPT = E // NW
CH = 80
NCHUNK = EPT // CH
NB = 2
ROWS_PT = NP // NS

_f32 = jnp.float32


def _proj_body(x_ref, w_ref, a_ref, h_ref, al_ref):
    h = jnp.dot(x_ref[...], w_ref[...], preferred_element_type=_f32)
    h_ref[...] = h
    al_ref[...] = jnp.dot(h, a_ref[...], preferred_element_type=_f32)


def _proj(x, w, a_pad, blk=1024):
    grid = NP // blk
    return pl.pallas_call(
        _proj_body,
        grid=(grid,),
        in_specs=[
            pl.BlockSpec((blk, D), lambda i: (i, 0)),
            pl.BlockSpec((D, D), lambda i: (0, 0)),
            pl.BlockSpec((D, 128), lambda i: (0, 0)),
        ],
        out_specs=[
            pl.BlockSpec((blk, D), lambda i: (i, 0)),
            pl.BlockSpec((blk, 128), lambda i: (i, 0)),
        ],
        out_shape=[
            jax.ShapeDtypeStruct((NP, D), _f32),
            jax.ShapeDtypeStruct((NP, 128), _f32),
        ],
    )(x, w, a_pad)


def _combine_body(p_ref, den_ref, o_ref):
    den = jnp.sum(den_ref[...], axis=0)
    inv = 1.0 / (den + 1e-16)
    o_ref[...] = jnp.maximum((p_ref[0] + p_ref[1]) * inv, 0.0)


def _combine(p, den, blk=1024):
    grid = NP // blk
    return pl.pallas_call(
        _combine_body,
        grid=(grid,),
        in_specs=[
            pl.BlockSpec((2, blk, D), lambda i: (0, i, 0)),
            pl.BlockSpec((NC, blk, 1), lambda i: (0, i, 0)),
        ],
        out_specs=pl.BlockSpec((blk, D), lambda i: (i, 0)),
        out_shape=jax.ShapeDtypeStruct((NP, D), _f32),
    )(p, den)


def _edge_body(src_hbm, dst_hbm, h_hbm, as_hbm, ad_hbm,
               outp_hbm, denp_hbm,
               as_v, ad_v, srcb, dstb, wb, rows_v,
               outp_s, den_s, gsem):
    c = lax.axis_index("c")
    s = lax.axis_index("s")
    wid = c * NS + s
    ebase = wid * EPT

    pltpu.sync_copy(as_hbm, as_v)
    pltpu.sync_copy(ad_hbm, ad_v)

    def _zr(i, _):
        for j in range(D // L):
            rows_v[0, i, pl.ds(j * L, L)] = jnp.zeros((L,), _f32)
        return 0
    lax.fori_loop(0, CH, _zr, 0)
    for g in range(CH // L):
        wb[0, pl.ds(g * L, L)] = jnp.zeros((L,), _f32)
    for k in range(ROWS_PT // CH):
        pltpu.sync_copy(rows_v.at[0],
                        outp_s.at[pl.ds(s * ROWS_PT + k * CH, CH), :])
        pltpu.sync_copy(wb.at[0], den_s.at[pl.ds(s * ROWS_PT + k * CH, CH)])
    plsc.subcore_barrier()

    def _fill_and_gather(jn, bn):
        base = ebase + jn * CH
        pltpu.sync_copy(src_hbm.at[pl.ds(base, CH)], srcb.at[bn])
        pltpu.sync_copy(dst_hbm.at[pl.ds(base, CH)], dstb.at[bn])
        pltpu.async_copy(h_hbm.at[srcb.at[bn]], rows_v.at[bn], gsem.at[bn])

    def _wait_gather(b):
        pltpu.make_async_copy(h_hbm.at[srcb.at[b]], rows_v.at[b],
                              gsem.at[b]).wait()

    def _compute_and_scatter(b):
        _wait_gather(b)
        for g2 in range(CH // L):
            si = srcb[b, pl.ds(g2 * L, L)]
            di = dstb[b, pl.ds(g2 * L, L)]
            e = plsc.load_gather(as_v, [si]) + plsc.load_gather(ad_v, [di])
            e = jnp.maximum(e, 0.2 * e)
            w = jnp.exp(e)
            wb[b, pl.ds(g2 * L, L)] = w
            for ei in range(L):
                r = g2 * L + ei
                ws = w[ei]
                for j2 in range(D // L):
                    sl = pl.ds(j2 * L, L)
                    rows_v[b, r, sl] = rows_v[b, r, sl] * ws
        pltpu.sync_copy(rows_v.at[b], outp_s.at[dstb.at[b]], add=True)
        pltpu.sync_copy(wb.at[b], den_s.at[dstb.at[b]], add=True)

    _fill_and_gather(0, 0)

    def _group(g, _):
        for b in range(NB):
            j = g * NB + b
            _fill_and_gather(j + 1, 1 - b)
            _compute_and_scatter(b)
        return 0
    lax.fori_loop(0, (NCHUNK - 1) // NB, _group, 0)
    _compute_and_scatter(0)
    plsc.subcore_barrier()

    rbase = s * ROWS_PT
    pltpu.sync_copy(outp_s.at[pl.ds(rbase, ROWS_PT), :],
                    outp_hbm.at[c, pl.ds(rbase, ROWS_PT), :])
    pltpu.sync_copy(den_s.at[pl.ds(rbase, ROWS_PT)],
                    denp_hbm.at[c, pl.ds(rbase, ROWS_PT)])


def _edge_pass(src, dst, h, a_s, a_d):
    mesh = plsc.VectorSubcoreMesh(core_axis_name="c", subcore_axis_name="s")
    kern = pl.kernel(
        _edge_body,
        out_type=[
            jax.ShapeDtypeStruct((NC, NP, D), _f32),
            jax.ShapeDtypeStruct((NC, NP), _f32),
        ],
        mesh=mesh,
        scratch_types=[
            pltpu.VMEM((NP,), _f32),
            pltpu.VMEM((NP,), _f32),
            pltpu.VMEM((NB, CH), jnp.int32),
            pltpu.VMEM((NB, CH), jnp.int32),
            pltpu.VMEM((NB, CH), _f32),
            pltpu.VMEM((NB, CH, D), _f32),
            pltpu.VMEM_SHARED((NP, D), _f32),
            pltpu.VMEM_SHARED((NP,), _f32),
            pltpu.SemaphoreType.DMA((NB,)),
        ],
        compiler_params=pltpu.CompilerParams(needs_layout_passes=False),
    )
    return kern(src, dst, h, a_s, a_d)


@jax.jit
def kernel(x, edge_index, W1, a_src1, a_dst1, W2, a_src2, a_dst2):
    src = edge_index[0]
    dst = edge_index[1]
    xp = jnp.pad(x, ((0, NP - N), (0, 0)))

    def a_pad(a_s, a_d):
        ap = jnp.zeros((D, 128), _f32)
        return ap.at[:, 0].set(a_s).at[:, 1].set(a_d)

    h1, al1 = _proj(xp, W1, a_pad(a_src1, a_dst1))
    p1, den1 = _edge_pass(src, dst, h1, al1[:, 0], al1[:, 1])
    y1 = _combine(p1, den1[:, :, None])
    h2, al2 = _proj(y1, W2, a_pad(a_src2, a_dst2))
    p2, den2 = _edge_pass(src, dst, h2, al2[:, 0], al2[:, 1])
    y2 = _combine(p2, den2[:, :, None])
    return y2[:N]
